# baseline (speedup 1.0000x reference)
import dataclasses
import functools

import jax
import jax.numpy as jnp
from jax import lax
from jax.experimental import pallas as pl
from jax.experimental.pallas import tpu as pltpu
from jax.experimental.pallas import tpu_sc as plsc

N = 10000
E = 320000
D = 128
B = 8192
NC = 2
NS = 16
LANES = 16
NT = NC * NS

NPAD = 10240
NODES_PER_TILE = NPAD // NS
ACC_ROWS_PER_TILE = NPAD // NS

EDGE_CHUNK = 128
EPAD = 327680
CHUNKS_PER_TILE = EPAD // (EDGE_CHUNK * NT)
IDX_PER_TILE = EPAD // NS

LAB_CHUNK = 128
LAB_CHUNKS_PER_TILE = B // (LAB_CHUNK * NT)

_mesh = plsc.VectorSubcoreMesh(core_axis_name="c", subcore_axis_name="s")

_sc_params = pltpu.CompilerParams()
if "needs_layout_passes" in pltpu.CompilerParams.__dataclass_fields__:
    _sc_params = dataclasses.replace(_sc_params, needs_layout_passes=False)


def _newton_rsqrt(m):
    xi = plsc.bitcast(m, jnp.int32)
    yi = jnp.int32(0x5F3759DF) - lax.shift_right_logical(xi, 1)
    y = plsc.bitcast(yi, jnp.float32)
    for _ in range(3):
        y = y * (1.5 - 0.5 * m * y * y)
    return y


@jax.jit
def _degree_norms(src, dst):
    @functools.partial(
        pl.kernel,
        out_type=(
            jax.ShapeDtypeStruct((NPAD,), jnp.float32),
            jax.ShapeDtypeStruct((NPAD,), jnp.float32),
        ),
        mesh=_mesh,
        compiler_params=_sc_params,
        scratch_types=[
            pltpu.VMEM((IDX_PER_TILE,), jnp.int32),
            pltpu.VMEM((NPAD,), jnp.float32),
            pltpu.VMEM((NS, NODES_PER_TILE), jnp.float32),
            pltpu.VMEM((NODES_PER_TILE,), jnp.float32),
            pltpu.VMEM_SHARED((NS, NPAD), jnp.float32),
        ],
    )
    def k(s_hbm, d_hbm, ons_hbm, ond_hbm, idx_v, hist_v, merge_v, norm_v, slab):
        c = lax.axis_index("c")
        s = lax.axis_index("s")

        @pl.when(c == 0)
        def _():
            pltpu.sync_copy(s_hbm.at[pl.ds(s * IDX_PER_TILE, IDX_PER_TILE)], idx_v)

        @pl.when(c == 1)
        def _():
            pltpu.sync_copy(d_hbm.at[pl.ds(s * IDX_PER_TILE, IDX_PER_TILE)], idx_v)

        zeros16 = jnp.zeros((LANES,), jnp.float32)

        @pl.loop(0, NPAD // LANES)
        def _(i):
            hist_v[pl.ds(i * LANES, LANES)] = zeros16

        ones16 = jnp.ones((LANES,), jnp.float32)

        @pl.loop(0, IDX_PER_TILE // LANES)
        def _(i):
            idx = idx_v[pl.ds(i * LANES, LANES)]
            plsc.addupdate_scatter(hist_v, [idx], ones16)

        pltpu.sync_copy(hist_v, slab.at[s])
        plsc.subcore_barrier()
        for t in range(NS):
            pltpu.sync_copy(
                slab.at[t, pl.ds(s * NODES_PER_TILE, NODES_PER_TILE)],
                merge_v.at[t],
            )

        @pl.loop(0, NODES_PER_TILE // LANES)
        def _(j):
            d = merge_v[0, pl.ds(j * LANES, LANES)]
            for t in range(1, NS):
                d = d + merge_v[t, pl.ds(j * LANES, LANES)]
            m = jnp.maximum(d, 1.0)
            r = _newton_rsqrt(m)
            norm_v[pl.ds(j * LANES, LANES)] = jnp.where(d > 0.0, r, 0.0)

        @pl.when(c == 0)
        def _():
            pltpu.sync_copy(
                norm_v, ons_hbm.at[pl.ds(s * NODES_PER_TILE, NODES_PER_TILE)]
            )

        @pl.when(c == 1)
        def _():
            pltpu.sync_copy(
                norm_v, ond_hbm.at[pl.ds(s * NODES_PER_TILE, NODES_PER_TILE)]
            )

    return k(src, dst)


@jax.jit
def _aggregate(h, sd):
    @functools.partial(
        pl.kernel,
        out_type=jax.ShapeDtypeStruct((NC, NPAD, D), jnp.float32),
        mesh=_mesh,
        compiler_params=_sc_params,
        scratch_types=[
            pltpu.VMEM((2, EDGE_CHUNK), jnp.int32),
            pltpu.VMEM((2, EDGE_CHUNK), jnp.int32),
            pltpu.VMEM((EDGE_CHUNK, D), jnp.float32),
            pltpu.VMEM((EDGE_CHUNK, D), jnp.float32),
            pltpu.VMEM((8, D), jnp.float32),
            pltpu.VMEM_SHARED((NPAD, D), jnp.float32),
            pltpu.SemaphoreType.DMA,
            pltpu.SemaphoreType.DMA,
            pltpu.SemaphoreType.DMA,
            pltpu.SemaphoreType.DMA,
        ],
    )
    def k(h_hbm, sd_hbm, out_hbm, sd0, sd1, rows0, rows1, zbuf, acc,
          semi0, semi1, semr0, semr1):
        c = lax.axis_index("c")
        s = lax.axis_index("s")
        t = c * NS + s

        zeros16 = jnp.zeros((LANES,), jnp.float32)

        @pl.loop(0, 8)
        def _(i):
            for j in range(D // LANES):
                zbuf[i, pl.ds(j * LANES, LANES)] = zeros16

        @pl.loop(0, ACC_ROWS_PER_TILE // 8)
        def _(q):
            pltpu.sync_copy(
                zbuf,
                acc.at[pl.ds(s * ACC_ROWS_PER_TILE + q * 8, 8)],
            )

        pltpu.async_copy(sd_hbm.at[t, 0], sd0, semi0)
        pltpu.async_copy(sd_hbm.at[t, 1], sd1, semi1)
        plsc.subcore_barrier()

        @pl.loop(0, CHUNKS_PER_TILE // 2 - 1)
        def _(j):
            i = j * 2
            pltpu.make_async_copy(sd_hbm.at[t, i], sd0, semi0).wait()
            pltpu.async_copy(h_hbm.at[sd0.at[0]], rows0, semr0).wait()
            pltpu.sync_copy(rows0, acc.at[sd0.at[1]], add=True)
            pltpu.async_copy(sd_hbm.at[t, i + 2], sd0, semi0)
            pltpu.make_async_copy(sd_hbm.at[t, i + 1], sd1, semi1).wait()
            pltpu.async_copy(h_hbm.at[sd1.at[0]], rows1, semr1).wait()
            pltpu.sync_copy(rows1, acc.at[sd1.at[1]], add=True)
            pltpu.async_copy(sd_hbm.at[t, i + 3], sd1, semi1)

        i = CHUNKS_PER_TILE - 2
        pltpu.make_async_copy(sd_hbm.at[t, i], sd0, semi0).wait()
        pltpu.async_copy(h_hbm.at[sd0.at[0]], rows0, semr0).wait()
        pltpu.sync_copy(rows0, acc.at[sd0.at[1]], add=True)
        pltpu.make_async_copy(sd_hbm.at[t, i + 1], sd1, semi1).wait()
        pltpu.async_copy(h_hbm.at[sd1.at[0]], rows1, semr1).wait()
        pltpu.sync_copy(rows1, acc.at[sd1.at[1]], add=True)

        plsc.subcore_barrier()
        pltpu.sync_copy(
            acc.at[pl.ds(s * ACC_ROWS_PER_TILE, ACC_ROWS_PER_TILE)],
            out_hbm.at[c, pl.ds(s * ACC_ROWS_PER_TILE, ACC_ROWS_PER_TILE)],
        )

    return k(h, sd)


@jax.jit
def _label_gather(h2, labr):
    @functools.partial(
        pl.kernel,
        out_type=jax.ShapeDtypeStruct((B, D), jnp.float32),
        mesh=_mesh,
        compiler_params=_sc_params,
        scratch_types=[
            pltpu.VMEM((LAB_CHUNKS_PER_TILE, LAB_CHUNK), jnp.int32),
            pltpu.VMEM((LAB_CHUNK, D), jnp.float32),
            pltpu.SemaphoreType.DMA,
        ],
    )
    def k(h_hbm, l_hbm, out_hbm, lab_v, rows_v, sem):
        c = lax.axis_index("c")
        s = lax.axis_index("s")
        t = c * NS + s
        pltpu.sync_copy(l_hbm.at[t], lab_v)
        for j in range(LAB_CHUNKS_PER_TILE):
            pltpu.async_copy(h_hbm.at[lab_v.at[j]], rows_v, sem).wait()
            pltpu.sync_copy(
                rows_v,
                out_hbm.at[
                    pl.ds((t * LAB_CHUNKS_PER_TILE + j) * LAB_CHUNK, LAB_CHUNK)
                ],
            )

    return k(h2, labr)


def _scale_body(f_ref, n_ref, o_ref):
    o_ref[...] = f_ref[...] * n_ref[...]


@jax.jit
def _scale(feat, ns_col):
    blk = 1024
    return pl.pallas_call(
        _scale_body,
        grid=(NPAD // blk,),
        in_specs=[
            pl.BlockSpec((blk, D), lambda i: (i, 0)),
            pl.BlockSpec((blk, 1), lambda i: (i, 0)),
        ],
        out_specs=pl.BlockSpec((blk, D), lambda i: (i, 0)),
        out_shape=jax.ShapeDtypeStruct((NPAD, D), jnp.float32),
    )(feat, ns_col)


def _mid_body(p_ref, nd_ref, ns_ref, w_ref, b_ref, o_ref):
    agg = p_ref[0] + p_ref[1]
    x = agg * nd_ref[...]
    y = jnp.dot(x, w_ref[...], preferred_element_type=jnp.float32) + b_ref[...]
    o_ref[...] = jnp.maximum(y, 0.0) * ns_ref[...]


@jax.jit
def _layer_mid(p, nd_col, ns_col, W, b_row):
    blk = 1024
    return pl.pallas_call(
        _mid_body,
        grid=(NPAD // blk,),
        in_specs=[
            pl.BlockSpec((NC, blk, D), lambda i: (0, i, 0)),
            pl.BlockSpec((blk, 1), lambda i: (i, 0)),
            pl.BlockSpec((blk, 1), lambda i: (i, 0)),
            pl.BlockSpec((D, D), lambda i: (0, 0)),
            pl.BlockSpec((1, D), lambda i: (0, 0)),
        ],
        out_specs=pl.BlockSpec((blk, D), lambda i: (i, 0)),
        out_shape=jax.ShapeDtypeStruct((NPAD, D), jnp.float32),
    )(p, nd_col, ns_col, W, b_row)


def _out_body(p_ref, nd_ref, w_ref, b_ref, o_ref):
    agg = p_ref[0] + p_ref[1]
    x = agg * nd_ref[...]
    o_ref[...] = (
        jnp.dot(x, w_ref[...], preferred_element_type=jnp.float32) + b_ref[...]
    )


@jax.jit
def _layer_out(p, nd_col, W, b_row):
    blk = 1024
    return pl.pallas_call(
        _out_body,
        grid=(NPAD // blk,),
        in_specs=[
            pl.BlockSpec((NC, blk, D), lambda i: (0, i, 0)),
            pl.BlockSpec((blk, 1), lambda i: (i, 0)),
            pl.BlockSpec((D, D), lambda i: (0, 0)),
            pl.BlockSpec((1, D), lambda i: (0, 0)),
        ],
        out_specs=pl.BlockSpec((blk, D), lambda i: (i, 0)),
        out_shape=jax.ShapeDtypeStruct((NPAD, D), jnp.float32),
    )(p, nd_col, W, b_row)


@jax.jit
def kernel(feat, edge_index, labels, W1, b1, W2, b2):
    ei = edge_index.astype(jnp.int32)
    pad = jnp.full((2, EPAD - E), NPAD - 1, jnp.int32)
    eip = jnp.concatenate([ei, pad], axis=1)
    src = eip[0]
    dst = eip[1]
    srcr = src.reshape(NT, CHUNKS_PER_TILE, 1, EDGE_CHUNK)
    dstr = dst.reshape(NT, CHUNKS_PER_TILE, 1, EDGE_CHUNK)
    sd = jnp.concatenate([srcr, dstr], axis=2)
    labr = labels.astype(jnp.int32).reshape(NT, LAB_CHUNKS_PER_TILE, LAB_CHUNK)
    featp = jnp.concatenate(
        [feat, jnp.zeros((NPAD - N, D), jnp.float32)], axis=0
    )

    norm_s, norm_d = _degree_norms(src, dst)
    ns_col = norm_s.reshape(NPAD, 1)
    nd_col = norm_d.reshape(NPAD, 1)

    h1s = _scale(featp, ns_col)
    p1 = _aggregate(h1s, sd)
    h2s = _layer_mid(p1, nd_col, ns_col, W1, b1.reshape(1, D))
    p2 = _aggregate(h2s, sd)
    h2 = _layer_out(p2, nd_col, W2, b2.reshape(1, D))
    return _label_gather(h2, labr)


# speedup vs baseline: 1.1726x; 1.1726x over previous
import dataclasses
import functools

import jax
import jax.numpy as jnp
from jax import lax
from jax.experimental import pallas as pl
from jax.experimental.pallas import tpu as pltpu
from jax.experimental.pallas import tpu_sc as plsc

N = 10000
E = 320000
D = 128
B = 8192
NC = 2
NS = 16
LANES = 16
NT = NC * NS

NPAD = 10240
NODES_PER_TILE = NPAD // NS
ACC_ROWS_PER_TILE = NPAD // NS

EDGE_CHUNK = 128
EPAD = 327680
CHUNKS_PER_TILE = EPAD // (EDGE_CHUNK * NT)
IDX_PER_TILE = EPAD // NS

LAB_CHUNK = 128
LAB_CHUNKS_PER_TILE = B // (LAB_CHUNK * NT)

_mesh = plsc.VectorSubcoreMesh(core_axis_name="c", subcore_axis_name="s")

_sc_params = pltpu.CompilerParams()
if "needs_layout_passes" in pltpu.CompilerParams.__dataclass_fields__:
    _sc_params = dataclasses.replace(_sc_params, needs_layout_passes=False)


def _newton_rsqrt(m):
    xi = plsc.bitcast(m, jnp.int32)
    yi = jnp.int32(0x5F3759DF) - lax.shift_right_logical(xi, 1)
    y = plsc.bitcast(yi, jnp.float32)
    for _ in range(3):
        y = y * (1.5 - 0.5 * m * y * y)
    return y


@jax.jit
def _degree_norms(src, dst):
    @functools.partial(
        pl.kernel,
        out_type=(
            jax.ShapeDtypeStruct((NPAD,), jnp.float32),
            jax.ShapeDtypeStruct((NPAD,), jnp.float32),
        ),
        mesh=_mesh,
        compiler_params=_sc_params,
        scratch_types=[
            pltpu.VMEM((IDX_PER_TILE,), jnp.int32),
            pltpu.VMEM((NPAD,), jnp.float32),
            pltpu.VMEM((NS, NODES_PER_TILE), jnp.float32),
            pltpu.VMEM((NODES_PER_TILE,), jnp.float32),
            pltpu.VMEM_SHARED((NS, NPAD), jnp.float32),
        ],
    )
    def k(s_hbm, d_hbm, ons_hbm, ond_hbm, idx_v, hist_v, merge_v, norm_v, slab):
        c = lax.axis_index("c")
        s = lax.axis_index("s")

        @pl.when(c == 0)
        def _():
            pltpu.sync_copy(s_hbm.at[pl.ds(s * IDX_PER_TILE, IDX_PER_TILE)], idx_v)

        @pl.when(c == 1)
        def _():
            pltpu.sync_copy(d_hbm.at[pl.ds(s * IDX_PER_TILE, IDX_PER_TILE)], idx_v)

        zeros16 = jnp.zeros((LANES,), jnp.float32)

        @pl.loop(0, NPAD // LANES)
        def _(i):
            hist_v[pl.ds(i * LANES, LANES)] = zeros16

        ones16 = jnp.ones((LANES,), jnp.float32)

        @pl.loop(0, IDX_PER_TILE // LANES)
        def _(i):
            idx = idx_v[pl.ds(i * LANES, LANES)]
            plsc.addupdate_scatter(hist_v, [idx], ones16)

        pltpu.sync_copy(hist_v, slab.at[s])
        plsc.subcore_barrier()
        for t in range(NS):
            pltpu.sync_copy(
                slab.at[t, pl.ds(s * NODES_PER_TILE, NODES_PER_TILE)],
                merge_v.at[t],
            )

        @pl.loop(0, NODES_PER_TILE // LANES)
        def _(j):
            d = merge_v[0, pl.ds(j * LANES, LANES)]
            for t in range(1, NS):
                d = d + merge_v[t, pl.ds(j * LANES, LANES)]
            m = jnp.maximum(d, 1.0)
            r = _newton_rsqrt(m)
            norm_v[pl.ds(j * LANES, LANES)] = jnp.where(d > 0.0, r, 0.0)

        @pl.when(c == 0)
        def _():
            pltpu.sync_copy(
                norm_v, ons_hbm.at[pl.ds(s * NODES_PER_TILE, NODES_PER_TILE)]
            )

        @pl.when(c == 1)
        def _():
            pltpu.sync_copy(
                norm_v, ond_hbm.at[pl.ds(s * NODES_PER_TILE, NODES_PER_TILE)]
            )

    return k(src, dst)


@jax.jit
def _aggregate(h, sd):
    HALF = CHUNKS_PER_TILE // 2

    @functools.partial(
        pl.kernel,
        out_type=jax.ShapeDtypeStruct((NC, NPAD, D), jnp.float32),
        mesh=_mesh,
        compiler_params=_sc_params,
        scratch_types=[
            pltpu.VMEM((HALF, EDGE_CHUNK), jnp.int32),
            pltpu.VMEM((HALF, EDGE_CHUNK), jnp.int32),
            pltpu.VMEM((EDGE_CHUNK, D), jnp.float32),
            pltpu.VMEM((EDGE_CHUNK, D), jnp.float32),
            pltpu.VMEM((8, D), jnp.float32),
            pltpu.VMEM_SHARED((NPAD, D), jnp.float32),
            pltpu.SemaphoreType.DMA,
            pltpu.SemaphoreType.DMA,
            pltpu.SemaphoreType.DMA,
            pltpu.SemaphoreType.DMA,
        ],
    )
    def k(h_hbm, sd_hbm, out_hbm, sidx, didx, rows0, rows1, zbuf, acc,
          semi0, semi1, semr0, semr1):
        c = lax.axis_index("c")
        s = lax.axis_index("s")
        t = c * NS + s

        zeros16 = jnp.zeros((LANES,), jnp.float32)

        @pl.loop(0, 8)
        def _(i):
            for j in range(D // LANES):
                zbuf[i, pl.ds(j * LANES, LANES)] = zeros16

        @pl.loop(0, ACC_ROWS_PER_TILE // 8)
        def _(q):
            pltpu.sync_copy(
                zbuf,
                acc.at[pl.ds(s * ACC_ROWS_PER_TILE + q * 8, 8)],
            )

        pltpu.async_copy(sd_hbm.at[t, 0, pl.ds(0, HALF)], sidx, semi0)
        pltpu.async_copy(sd_hbm.at[t, 1, pl.ds(0, HALF)], didx, semi1)
        plsc.subcore_barrier()

        def do_half(base):
            pltpu.make_async_copy(
                sd_hbm.at[t, 0, pl.ds(base, HALF)], sidx, semi0
            ).wait()
            pltpu.make_async_copy(
                sd_hbm.at[t, 1, pl.ds(base, HALF)], didx, semi1
            ).wait()
            pltpu.async_copy(h_hbm.at[sidx.at[0]], rows0, semr0)

            @pl.loop(0, HALF // 2 - 1)
            def _(j):
                i = j * 2
                pltpu.async_copy(h_hbm.at[sidx.at[i + 1]], rows1, semr1)
                pltpu.make_async_copy(h_hbm.at[sidx.at[i]], rows0, semr0).wait()
                pltpu.sync_copy(rows0, acc.at[didx.at[i]], add=True)
                pltpu.async_copy(h_hbm.at[sidx.at[i + 2]], rows0, semr0)
                pltpu.make_async_copy(
                    h_hbm.at[sidx.at[i + 1]], rows1, semr1
                ).wait()
                pltpu.sync_copy(rows1, acc.at[didx.at[i + 1]], add=True)

            last = HALF - 1
            pltpu.async_copy(h_hbm.at[sidx.at[last]], rows1, semr1)
            pltpu.make_async_copy(h_hbm.at[sidx.at[last - 1]], rows0, semr0).wait()
            pltpu.sync_copy(rows0, acc.at[didx.at[last - 1]], add=True)
            pltpu.make_async_copy(h_hbm.at[sidx.at[last]], rows1, semr1).wait()
            pltpu.sync_copy(rows1, acc.at[didx.at[last]], add=True)

        do_half(0)
        pltpu.async_copy(sd_hbm.at[t, 0, pl.ds(HALF, HALF)], sidx, semi0)
        pltpu.async_copy(sd_hbm.at[t, 1, pl.ds(HALF, HALF)], didx, semi1)
        do_half(HALF)

        plsc.subcore_barrier()
        pltpu.sync_copy(
            acc.at[pl.ds(s * ACC_ROWS_PER_TILE, ACC_ROWS_PER_TILE)],
            out_hbm.at[c, pl.ds(s * ACC_ROWS_PER_TILE, ACC_ROWS_PER_TILE)],
        )

    return k(h, sd)


@jax.jit
def _label_gather(h2, labr):
    @functools.partial(
        pl.kernel,
        out_type=jax.ShapeDtypeStruct((B, D), jnp.float32),
        mesh=_mesh,
        compiler_params=_sc_params,
        scratch_types=[
            pltpu.VMEM((LAB_CHUNKS_PER_TILE, LAB_CHUNK), jnp.int32),
            pltpu.VMEM((LAB_CHUNK, D), jnp.float32),
            pltpu.SemaphoreType.DMA,
        ],
    )
    def k(h_hbm, l_hbm, out_hbm, lab_v, rows_v, sem):
        c = lax.axis_index("c")
        s = lax.axis_index("s")
        t = c * NS + s
        pltpu.sync_copy(l_hbm.at[t], lab_v)
        for j in range(LAB_CHUNKS_PER_TILE):
            pltpu.async_copy(h_hbm.at[lab_v.at[j]], rows_v, sem).wait()
            pltpu.sync_copy(
                rows_v,
                out_hbm.at[
                    pl.ds((t * LAB_CHUNKS_PER_TILE + j) * LAB_CHUNK, LAB_CHUNK)
                ],
            )

    return k(h2, labr)


def _scale_body(f_ref, n_ref, o_ref):
    o_ref[...] = f_ref[...] * n_ref[...]


@jax.jit
def _scale(feat, ns_col):
    blk = 1024
    return pl.pallas_call(
        _scale_body,
        grid=(NPAD // blk,),
        in_specs=[
            pl.BlockSpec((blk, D), lambda i: (i, 0)),
            pl.BlockSpec((blk, 1), lambda i: (i, 0)),
        ],
        out_specs=pl.BlockSpec((blk, D), lambda i: (i, 0)),
        out_shape=jax.ShapeDtypeStruct((NPAD, D), jnp.float32),
    )(feat, ns_col)


def _mid_body(p_ref, nd_ref, ns_ref, w_ref, b_ref, o_ref):
    agg = p_ref[0] + p_ref[1]
    x = agg * nd_ref[...]
    y = jnp.dot(x, w_ref[...], preferred_element_type=jnp.float32) + b_ref[...]
    o_ref[...] = jnp.maximum(y, 0.0) * ns_ref[...]


@jax.jit
def _layer_mid(p, nd_col, ns_col, W, b_row):
    blk = 1024
    return pl.pallas_call(
        _mid_body,
        grid=(NPAD // blk,),
        in_specs=[
            pl.BlockSpec((NC, blk, D), lambda i: (0, i, 0)),
            pl.BlockSpec((blk, 1), lambda i: (i, 0)),
            pl.BlockSpec((blk, 1), lambda i: (i, 0)),
            pl.BlockSpec((D, D), lambda i: (0, 0)),
            pl.BlockSpec((1, D), lambda i: (0, 0)),
        ],
        out_specs=pl.BlockSpec((blk, D), lambda i: (i, 0)),
        out_shape=jax.ShapeDtypeStruct((NPAD, D), jnp.float32),
    )(p, nd_col, ns_col, W, b_row)


def _out_body(p_ref, nd_ref, w_ref, b_ref, o_ref):
    agg = p_ref[0] + p_ref[1]
    x = agg * nd_ref[...]
    o_ref[...] = (
        jnp.dot(x, w_ref[...], preferred_element_type=jnp.float32) + b_ref[...]
    )


@jax.jit
def _layer_out(p, nd_col, W, b_row):
    blk = 1024
    return pl.pallas_call(
        _out_body,
        grid=(NPAD // blk,),
        in_specs=[
            pl.BlockSpec((NC, blk, D), lambda i: (0, i, 0)),
            pl.BlockSpec((blk, 1), lambda i: (i, 0)),
            pl.BlockSpec((D, D), lambda i: (0, 0)),
            pl.BlockSpec((1, D), lambda i: (0, 0)),
        ],
        out_specs=pl.BlockSpec((blk, D), lambda i: (i, 0)),
        out_shape=jax.ShapeDtypeStruct((NPAD, D), jnp.float32),
    )(p, nd_col, W, b_row)


@jax.jit
def kernel(feat, edge_index, labels, W1, b1, W2, b2):
    ei = edge_index.astype(jnp.int32)
    pad = jnp.full((2, EPAD - E), NPAD - 1, jnp.int32)
    eip = jnp.concatenate([ei, pad], axis=1)
    src = eip[0]
    dst = eip[1]
    srcr = src.reshape(NT, 1, CHUNKS_PER_TILE, EDGE_CHUNK)
    dstr = dst.reshape(NT, 1, CHUNKS_PER_TILE, EDGE_CHUNK)
    sd = jnp.concatenate([srcr, dstr], axis=1)
    labr = labels.astype(jnp.int32).reshape(NT, LAB_CHUNKS_PER_TILE, LAB_CHUNK)
    featp = jnp.concatenate(
        [feat, jnp.zeros((NPAD - N, D), jnp.float32)], axis=0
    )

    norm_s, norm_d = _degree_norms(src, dst)
    ns_col = norm_s.reshape(NPAD, 1)
    nd_col = norm_d.reshape(NPAD, 1)

    h1s = _scale(featp, ns_col)
    p1 = _aggregate(h1s, sd)
    h2s = _layer_mid(p1, nd_col, ns_col, W1, b1.reshape(1, D))
    p2 = _aggregate(h2s, sd)
    h2 = _layer_out(p2, nd_col, W2, b2.reshape(1, D))
    return _label_gather(h2, labr)


# speedup vs baseline: 3.1945x; 2.7243x over previous
import dataclasses
import functools

import jax
import jax.numpy as jnp
from jax import lax
from jax.experimental import pallas as pl
from jax.experimental.pallas import tpu as pltpu
from jax.experimental.pallas import tpu_sc as plsc

N = 10000
E = 320000
D = 128
B = 8192
NC = 2
NS = 16
LANES = 16
NT = NC * NS

NPAD = 10240
NODES_PER_TILE = NPAD // NS
ACC_ROWS_PER_TILE = NPAD // NS

EDGE_CHUNK = 128
EPAD = 327680
CHUNKS_PER_TILE = EPAD // (EDGE_CHUNK * NT)
IDX_PER_TILE = EPAD // NS

LAB_CHUNK = 128
LAB_CHUNKS_PER_TILE = B // (LAB_CHUNK * NT)

_mesh = plsc.VectorSubcoreMesh(core_axis_name="c", subcore_axis_name="s")

_sc_params = pltpu.CompilerParams()
if "needs_layout_passes" in pltpu.CompilerParams.__dataclass_fields__:
    _sc_params = dataclasses.replace(_sc_params, needs_layout_passes=False)


def _newton_rsqrt(m):
    xi = plsc.bitcast(m, jnp.int32)
    yi = jnp.int32(0x5F3759DF) - lax.shift_right_logical(xi, 1)
    y = plsc.bitcast(yi, jnp.float32)
    for _ in range(3):
        y = y * (1.5 - 0.5 * m * y * y)
    return y


@jax.jit
def _degree_norms(src, dst):
    @functools.partial(
        pl.kernel,
        out_type=(
            jax.ShapeDtypeStruct((NPAD,), jnp.float32),
            jax.ShapeDtypeStruct((NPAD,), jnp.float32),
        ),
        mesh=_mesh,
        compiler_params=_sc_params,
        scratch_types=[
            pltpu.VMEM((IDX_PER_TILE,), jnp.int32),
            pltpu.VMEM((NPAD,), jnp.float32),
            pltpu.VMEM((NS, NODES_PER_TILE), jnp.float32),
            pltpu.VMEM((NODES_PER_TILE,), jnp.float32),
            pltpu.VMEM_SHARED((NS, NPAD), jnp.float32),
        ],
    )
    def k(s_hbm, d_hbm, ons_hbm, ond_hbm, idx_v, hist_v, merge_v, norm_v, slab):
        c = lax.axis_index("c")
        s = lax.axis_index("s")

        @pl.when(c == 0)
        def _():
            pltpu.sync_copy(s_hbm.at[pl.ds(s * IDX_PER_TILE, IDX_PER_TILE)], idx_v)

        @pl.when(c == 1)
        def _():
            pltpu.sync_copy(d_hbm.at[pl.ds(s * IDX_PER_TILE, IDX_PER_TILE)], idx_v)

        zeros16 = jnp.zeros((LANES,), jnp.float32)

        @pl.loop(0, NPAD // LANES)
        def _(i):
            hist_v[pl.ds(i * LANES, LANES)] = zeros16

        ones16 = jnp.ones((LANES,), jnp.float32)

        @pl.loop(0, IDX_PER_TILE // LANES)
        def _(i):
            idx = idx_v[pl.ds(i * LANES, LANES)]
            plsc.addupdate_scatter(hist_v, [idx], ones16)

        pltpu.sync_copy(hist_v, slab.at[s])
        plsc.subcore_barrier()
        for t in range(NS):
            pltpu.sync_copy(
                slab.at[t, pl.ds(s * NODES_PER_TILE, NODES_PER_TILE)],
                merge_v.at[t],
            )

        @pl.loop(0, NODES_PER_TILE // LANES)
        def _(j):
            d = merge_v[0, pl.ds(j * LANES, LANES)]
            for t in range(1, NS):
                d = d + merge_v[t, pl.ds(j * LANES, LANES)]
            m = jnp.maximum(d, 1.0)
            r = _newton_rsqrt(m)
            norm_v[pl.ds(j * LANES, LANES)] = jnp.where(d > 0.0, r, 0.0)

        @pl.when(c == 0)
        def _():
            pltpu.sync_copy(
                norm_v, ons_hbm.at[pl.ds(s * NODES_PER_TILE, NODES_PER_TILE)]
            )

        @pl.when(c == 1)
        def _():
            pltpu.sync_copy(
                norm_v, ond_hbm.at[pl.ds(s * NODES_PER_TILE, NODES_PER_TILE)]
            )

    return k(src, dst)


@jax.jit
def _aggregate(h, sd):
    HALF = CHUNKS_PER_TILE // 2

    @functools.partial(
        pl.kernel,
        out_type=jax.ShapeDtypeStruct((NC, NPAD, D), jnp.float32),
        mesh=_mesh,
        compiler_params=_sc_params,
        scratch_types=[
            pltpu.VMEM((HALF, EDGE_CHUNK), jnp.int32),
            pltpu.VMEM((HALF, EDGE_CHUNK), jnp.int32),
            pltpu.VMEM((EDGE_CHUNK, D), jnp.float32),
            pltpu.VMEM((EDGE_CHUNK, D), jnp.float32),
            pltpu.VMEM((8, D), jnp.float32),
            pltpu.VMEM_SHARED((NPAD, D), jnp.float32),
            pltpu.SemaphoreType.DMA,
            pltpu.SemaphoreType.DMA,
            pltpu.SemaphoreType.DMA,
            pltpu.SemaphoreType.DMA,
        ],
    )
    def k(h_hbm, sd_hbm, out_hbm, sidx, didx, rows0, rows1, zbuf, acc,
          semi0, semi1, semr0, semr1):
        c = lax.axis_index("c")
        s = lax.axis_index("s")
        t = c * NS + s

        zeros16 = jnp.zeros((LANES,), jnp.float32)

        @pl.loop(0, 8)
        def _(i):
            for j in range(D // LANES):
                zbuf[i, pl.ds(j * LANES, LANES)] = zeros16

        @pl.loop(0, ACC_ROWS_PER_TILE // 8)
        def _(q):
            pltpu.sync_copy(
                zbuf,
                acc.at[pl.ds(s * ACC_ROWS_PER_TILE + q * 8, 8)],
            )

        pltpu.async_copy(sd_hbm.at[t, 0, pl.ds(0, HALF)], sidx, semi0)
        pltpu.async_copy(sd_hbm.at[t, 1, pl.ds(0, HALF)], didx, semi1)
        plsc.subcore_barrier()

        def do_half(base):
            pltpu.make_async_copy(
                sd_hbm.at[t, 0, pl.ds(base, HALF)], sidx, semi0
            ).wait()
            pltpu.make_async_copy(
                sd_hbm.at[t, 1, pl.ds(base, HALF)], didx, semi1
            ).wait()
            pltpu.async_copy(h_hbm.at[sidx.at[0]], rows0, semr0)

            @pl.loop(0, HALF // 2 - 1)
            def _(j):
                i = j * 2
                pltpu.async_copy(h_hbm.at[sidx.at[i + 1]], rows1, semr1)
                pltpu.make_async_copy(h_hbm.at[sidx.at[i]], rows0, semr0).wait()
                pltpu.sync_copy(rows0, acc.at[didx.at[i]], add=True)
                pltpu.async_copy(h_hbm.at[sidx.at[i + 2]], rows0, semr0)
                pltpu.make_async_copy(
                    h_hbm.at[sidx.at[i + 1]], rows1, semr1
                ).wait()
                pltpu.sync_copy(rows1, acc.at[didx.at[i + 1]], add=True)

            last = HALF - 1
            pltpu.async_copy(h_hbm.at[sidx.at[last]], rows1, semr1)
            pltpu.make_async_copy(h_hbm.at[sidx.at[last - 1]], rows0, semr0).wait()
            pltpu.sync_copy(rows0, acc.at[didx.at[last - 1]], add=True)
            pltpu.make_async_copy(h_hbm.at[sidx.at[last]], rows1, semr1).wait()
            pltpu.sync_copy(rows1, acc.at[didx.at[last]], add=True)

        do_half(0)
        pltpu.async_copy(sd_hbm.at[t, 0, pl.ds(HALF, HALF)], sidx, semi0)
        pltpu.async_copy(sd_hbm.at[t, 1, pl.ds(HALF, HALF)], didx, semi1)
        do_half(HALF)

        plsc.subcore_barrier()
        pltpu.sync_copy(
            acc.at[pl.ds(s * ACC_ROWS_PER_TILE, ACC_ROWS_PER_TILE)],
            out_hbm.at[c, pl.ds(s * ACC_ROWS_PER_TILE, ACC_ROWS_PER_TILE)],
        )

    return k(h, sd)


@jax.jit
def _label_gather(h2, labr):
    @functools.partial(
        pl.kernel,
        out_type=jax.ShapeDtypeStruct((B, D), jnp.float32),
        mesh=_mesh,
        compiler_params=_sc_params,
        scratch_types=[
            pltpu.VMEM((LAB_CHUNKS_PER_TILE, LAB_CHUNK), jnp.int32),
            pltpu.VMEM((LAB_CHUNK, D), jnp.float32),
            pltpu.SemaphoreType.DMA,
        ],
    )
    def k(h_hbm, l_hbm, out_hbm, lab_v, rows_v, sem):
        c = lax.axis_index("c")
        s = lax.axis_index("s")
        t = c * NS + s
        pltpu.sync_copy(l_hbm.at[t], lab_v)
        for j in range(LAB_CHUNKS_PER_TILE):
            pltpu.async_copy(h_hbm.at[lab_v.at[j]], rows_v, sem).wait()
            pltpu.sync_copy(
                rows_v,
                out_hbm.at[
                    pl.ds((t * LAB_CHUNKS_PER_TILE + j) * LAB_CHUNK, LAB_CHUNK)
                ],
            )

    return k(h2, labr)


def _scale_body(f_ref, n_ref, o_ref):
    o_ref[...] = f_ref[...] * n_ref[...]


@jax.jit
def _scale(feat, ns_col):
    blk = 1024
    return pl.pallas_call(
        _scale_body,
        grid=(NPAD // blk,),
        in_specs=[
            pl.BlockSpec((blk, D), lambda i: (i, 0)),
            pl.BlockSpec((blk, 1), lambda i: (i, 0)),
        ],
        out_specs=pl.BlockSpec((blk, D), lambda i: (i, 0)),
        out_shape=jax.ShapeDtypeStruct((NPAD, D), jnp.float32),
    )(feat, ns_col)


def _mid_body(p_ref, nd_ref, ns_ref, w_ref, b_ref, o_ref):
    agg = p_ref[0] + p_ref[1]
    x = agg * nd_ref[...]
    y = jnp.dot(x, w_ref[...], preferred_element_type=jnp.float32) + b_ref[...]
    o_ref[...] = jnp.maximum(y, 0.0) * ns_ref[...]


@jax.jit
def _layer_mid(p, nd_col, ns_col, W, b_row):
    blk = 1024
    return pl.pallas_call(
        _mid_body,
        grid=(NPAD // blk,),
        in_specs=[
            pl.BlockSpec((NC, blk, D), lambda i: (0, i, 0)),
            pl.BlockSpec((blk, 1), lambda i: (i, 0)),
            pl.BlockSpec((blk, 1), lambda i: (i, 0)),
            pl.BlockSpec((D, D), lambda i: (0, 0)),
            pl.BlockSpec((1, D), lambda i: (0, 0)),
        ],
        out_specs=pl.BlockSpec((blk, D), lambda i: (i, 0)),
        out_shape=jax.ShapeDtypeStruct((NPAD, D), jnp.float32),
    )(p, nd_col, ns_col, W, b_row)


def _out_body(p_ref, nd_ref, w_ref, b_ref, o_ref):
    agg = p_ref[0] + p_ref[1]
    x = agg * nd_ref[...]
    o_ref[...] = (
        jnp.dot(x, w_ref[...], preferred_element_type=jnp.float32) + b_ref[...]
    )


@jax.jit
def _layer_out(p, nd_col, W, b_row):
    blk = 1024
    return pl.pallas_call(
        _out_body,
        grid=(NPAD // blk,),
        in_specs=[
            pl.BlockSpec((NC, blk, D), lambda i: (0, i, 0)),
            pl.BlockSpec((blk, 1), lambda i: (i, 0)),
            pl.BlockSpec((D, D), lambda i: (0, 0)),
            pl.BlockSpec((1, D), lambda i: (0, 0)),
        ],
        out_specs=pl.BlockSpec((blk, D), lambda i: (i, 0)),
        out_shape=jax.ShapeDtypeStruct((NPAD, D), jnp.float32),
    )(p, nd_col, W, b_row)


@jax.jit
def kernel(feat, edge_index, labels, W1, b1, W2, b2):
    ei = edge_index.astype(jnp.int32)
    padidx = N + jnp.arange(EPAD - E, dtype=jnp.int32) % (NPAD - N)
    pad = jnp.stack([padidx, padidx])
    eip = jnp.concatenate([ei, pad], axis=1)
    src = eip[0]
    dst = eip[1]
    srcr = src.reshape(NT, 1, CHUNKS_PER_TILE, EDGE_CHUNK)
    dstr = dst.reshape(NT, 1, CHUNKS_PER_TILE, EDGE_CHUNK)
    sd = jnp.concatenate([srcr, dstr], axis=1)
    labr = labels.astype(jnp.int32).reshape(NT, LAB_CHUNKS_PER_TILE, LAB_CHUNK)
    featp = jnp.concatenate(
        [feat, jnp.zeros((NPAD - N, D), jnp.float32)], axis=0
    )

    norm_s, norm_d = _degree_norms(src, dst)
    ns_col = norm_s.reshape(NPAD, 1)
    nd_col = norm_d.reshape(NPAD, 1)

    h1s = _scale(featp, ns_col)
    p1 = _aggregate(h1s, sd)
    h2s = _layer_mid(p1, nd_col, ns_col, W1, b1.reshape(1, D))
    p2 = _aggregate(h2s, sd)
    h2 = _layer_out(p2, nd_col, W2, b2.reshape(1, D))
    return _label_gather(h2, labr)


# speedup vs baseline: 3.1985x; 1.0013x over previous
import dataclasses
import functools

import jax
import jax.numpy as jnp
from jax import lax
from jax.experimental import pallas as pl
from jax.experimental.pallas import tpu as pltpu
from jax.experimental.pallas import tpu_sc as plsc

N = 10000
E = 320000
D = 128
B = 8192
NC = 2
NS = 16
LANES = 16
NT = NC * NS

NPAD = 10240
NODES_PER_TILE = NPAD // NS
ACC_ROWS_PER_TILE = NPAD // NS

EDGE_CHUNK = 128
EPAD = 327680
CHUNKS_PER_TILE = EPAD // (EDGE_CHUNK * NT)
IDX_PER_TILE = EPAD // NS

LAB_CHUNK = 128
LAB_CHUNKS_PER_TILE = B // (LAB_CHUNK * NT)

_mesh = plsc.VectorSubcoreMesh(core_axis_name="c", subcore_axis_name="s")

_sc_params = pltpu.CompilerParams()
if "needs_layout_passes" in pltpu.CompilerParams.__dataclass_fields__:
    _sc_params = dataclasses.replace(_sc_params, needs_layout_passes=False)


def _newton_rsqrt(m):
    xi = plsc.bitcast(m, jnp.int32)
    yi = jnp.int32(0x5F3759DF) - lax.shift_right_logical(xi, 1)
    y = plsc.bitcast(yi, jnp.float32)
    for _ in range(3):
        y = y * (1.5 - 0.5 * m * y * y)
    return y


@jax.jit
def _degree_norms(src, dst):
    @functools.partial(
        pl.kernel,
        out_type=(
            jax.ShapeDtypeStruct((NPAD,), jnp.float32),
            jax.ShapeDtypeStruct((NPAD,), jnp.float32),
        ),
        mesh=_mesh,
        compiler_params=_sc_params,
        scratch_types=[
            pltpu.VMEM((IDX_PER_TILE,), jnp.int32),
            pltpu.VMEM((NPAD,), jnp.float32),
            pltpu.VMEM((NS, NODES_PER_TILE), jnp.float32),
            pltpu.VMEM((NODES_PER_TILE,), jnp.float32),
            pltpu.VMEM_SHARED((NS, NPAD), jnp.float32),
        ],
    )
    def k(s_hbm, d_hbm, ons_hbm, ond_hbm, idx_v, hist_v, merge_v, norm_v, slab):
        c = lax.axis_index("c")
        s = lax.axis_index("s")

        @pl.when(c == 0)
        def _():
            pltpu.sync_copy(s_hbm.at[pl.ds(s * IDX_PER_TILE, IDX_PER_TILE)], idx_v)

        @pl.when(c == 1)
        def _():
            pltpu.sync_copy(d_hbm.at[pl.ds(s * IDX_PER_TILE, IDX_PER_TILE)], idx_v)

        zeros16 = jnp.zeros((LANES,), jnp.float32)

        @pl.loop(0, NPAD // LANES)
        def _(i):
            hist_v[pl.ds(i * LANES, LANES)] = zeros16

        ones16 = jnp.ones((LANES,), jnp.float32)

        @pl.loop(0, IDX_PER_TILE // LANES)
        def _(i):
            idx = idx_v[pl.ds(i * LANES, LANES)]
            plsc.addupdate_scatter(hist_v, [idx], ones16)

        pltpu.sync_copy(hist_v, slab.at[s])
        plsc.subcore_barrier()
        for t in range(NS):
            pltpu.sync_copy(
                slab.at[t, pl.ds(s * NODES_PER_TILE, NODES_PER_TILE)],
                merge_v.at[t],
            )

        @pl.loop(0, NODES_PER_TILE // LANES)
        def _(j):
            d = merge_v[0, pl.ds(j * LANES, LANES)]
            for t in range(1, NS):
                d = d + merge_v[t, pl.ds(j * LANES, LANES)]
            m = jnp.maximum(d, 1.0)
            r = _newton_rsqrt(m)
            norm_v[pl.ds(j * LANES, LANES)] = jnp.where(d > 0.0, r, 0.0)

        @pl.when(c == 0)
        def _():
            pltpu.sync_copy(
                norm_v, ons_hbm.at[pl.ds(s * NODES_PER_TILE, NODES_PER_TILE)]
            )

        @pl.when(c == 1)
        def _():
            pltpu.sync_copy(
                norm_v, ond_hbm.at[pl.ds(s * NODES_PER_TILE, NODES_PER_TILE)]
            )

    return k(src, dst)


@jax.jit
def _aggregate(h, sd, zrows):
    HALF = CHUNKS_PER_TILE // 2

    @functools.partial(
        pl.kernel,
        out_type=jax.ShapeDtypeStruct((NC, NPAD, D), jnp.float32),
        mesh=_mesh,
        compiler_params=_sc_params,
        scratch_types=[
            pltpu.VMEM((HALF, EDGE_CHUNK), jnp.int32),
            pltpu.VMEM((HALF, EDGE_CHUNK), jnp.int32),
            pltpu.VMEM((EDGE_CHUNK, D), jnp.float32),
            pltpu.VMEM((EDGE_CHUNK, D), jnp.float32),
            pltpu.VMEM_SHARED((NPAD, D), jnp.float32),
            pltpu.SemaphoreType.DMA,
            pltpu.SemaphoreType.DMA,
            pltpu.SemaphoreType.DMA,
            pltpu.SemaphoreType.DMA,
            pltpu.SemaphoreType.DMA,
        ],
    )
    def k(h_hbm, sd_hbm, z_hbm, out_hbm, sidx, didx, rows0, rows1, acc,
          semi0, semi1, semr0, semr1, semz):
        c = lax.axis_index("c")
        s = lax.axis_index("s")
        t = c * NS + s

        pltpu.async_copy(
            z_hbm, acc.at[pl.ds(s * ACC_ROWS_PER_TILE, ACC_ROWS_PER_TILE)], semz
        )
        pltpu.async_copy(sd_hbm.at[t, 0, pl.ds(0, HALF)], sidx, semi0)
        pltpu.async_copy(sd_hbm.at[t, 1, pl.ds(0, HALF)], didx, semi1)
        pltpu.make_async_copy(
            z_hbm, acc.at[pl.ds(s * ACC_ROWS_PER_TILE, ACC_ROWS_PER_TILE)], semz
        ).wait()
        plsc.subcore_barrier()

        def do_half(base):
            pltpu.make_async_copy(
                sd_hbm.at[t, 0, pl.ds(base, HALF)], sidx, semi0
            ).wait()
            pltpu.make_async_copy(
                sd_hbm.at[t, 1, pl.ds(base, HALF)], didx, semi1
            ).wait()
            pltpu.async_copy(h_hbm.at[sidx.at[0]], rows0, semr0)

            @pl.loop(0, HALF // 2 - 1)
            def _(j):
                i = j * 2
                pltpu.async_copy(h_hbm.at[sidx.at[i + 1]], rows1, semr1)
                pltpu.make_async_copy(h_hbm.at[sidx.at[i]], rows0, semr0).wait()
                pltpu.sync_copy(rows0, acc.at[didx.at[i]], add=True)
                pltpu.async_copy(h_hbm.at[sidx.at[i + 2]], rows0, semr0)
                pltpu.make_async_copy(
                    h_hbm.at[sidx.at[i + 1]], rows1, semr1
                ).wait()
                pltpu.sync_copy(rows1, acc.at[didx.at[i + 1]], add=True)

            last = HALF - 1
            pltpu.async_copy(h_hbm.at[sidx.at[last]], rows1, semr1)
            pltpu.make_async_copy(h_hbm.at[sidx.at[last - 1]], rows0, semr0).wait()
            pltpu.sync_copy(rows0, acc.at[didx.at[last - 1]], add=True)
            pltpu.make_async_copy(h_hbm.at[sidx.at[last]], rows1, semr1).wait()
            pltpu.sync_copy(rows1, acc.at[didx.at[last]], add=True)

        do_half(0)
        pltpu.async_copy(sd_hbm.at[t, 0, pl.ds(HALF, HALF)], sidx, semi0)
        pltpu.async_copy(sd_hbm.at[t, 1, pl.ds(HALF, HALF)], didx, semi1)
        do_half(HALF)

        plsc.subcore_barrier()
        pltpu.sync_copy(
            acc.at[pl.ds(s * ACC_ROWS_PER_TILE, ACC_ROWS_PER_TILE)],
            out_hbm.at[c, pl.ds(s * ACC_ROWS_PER_TILE, ACC_ROWS_PER_TILE)],
        )

    return k(h, sd, zrows)


@jax.jit
def _label_gather(h2, labr):
    @functools.partial(
        pl.kernel,
        out_type=jax.ShapeDtypeStruct((B, D), jnp.float32),
        mesh=_mesh,
        compiler_params=_sc_params,
        scratch_types=[
            pltpu.VMEM((LAB_CHUNKS_PER_TILE, LAB_CHUNK), jnp.int32),
            pltpu.VMEM((LAB_CHUNK, D), jnp.float32),
            pltpu.SemaphoreType.DMA,
        ],
    )
    def k(h_hbm, l_hbm, out_hbm, lab_v, rows_v, sem):
        c = lax.axis_index("c")
        s = lax.axis_index("s")
        t = c * NS + s
        pltpu.sync_copy(l_hbm.at[t], lab_v)
        for j in range(LAB_CHUNKS_PER_TILE):
            pltpu.async_copy(h_hbm.at[lab_v.at[j]], rows_v, sem).wait()
            pltpu.sync_copy(
                rows_v,
                out_hbm.at[
                    pl.ds((t * LAB_CHUNKS_PER_TILE + j) * LAB_CHUNK, LAB_CHUNK)
                ],
            )

    return k(h2, labr)


def _scale_body(f_ref, n_ref, o_ref):
    o_ref[...] = f_ref[...] * n_ref[...]


@jax.jit
def _scale(feat, ns_col):
    blk = 1024
    return pl.pallas_call(
        _scale_body,
        grid=(NPAD // blk,),
        in_specs=[
            pl.BlockSpec((blk, D), lambda i: (i, 0)),
            pl.BlockSpec((blk, 1), lambda i: (i, 0)),
        ],
        out_specs=pl.BlockSpec((blk, D), lambda i: (i, 0)),
        out_shape=jax.ShapeDtypeStruct((NPAD, D), jnp.float32),
    )(feat, ns_col)


def _mid_body(p_ref, nd_ref, ns_ref, w_ref, b_ref, o_ref):
    agg = p_ref[0] + p_ref[1]
    x = agg * nd_ref[...]
    y = jnp.dot(x, w_ref[...], preferred_element_type=jnp.float32) + b_ref[...]
    o_ref[...] = jnp.maximum(y, 0.0) * ns_ref[...]


@jax.jit
def _layer_mid(p, nd_col, ns_col, W, b_row):
    blk = 1024
    return pl.pallas_call(
        _mid_body,
        grid=(NPAD // blk,),
        in_specs=[
            pl.BlockSpec((NC, blk, D), lambda i: (0, i, 0)),
            pl.BlockSpec((blk, 1), lambda i: (i, 0)),
            pl.BlockSpec((blk, 1), lambda i: (i, 0)),
            pl.BlockSpec((D, D), lambda i: (0, 0)),
            pl.BlockSpec((1, D), lambda i: (0, 0)),
        ],
        out_specs=pl.BlockSpec((blk, D), lambda i: (i, 0)),
        out_shape=jax.ShapeDtypeStruct((NPAD, D), jnp.float32),
    )(p, nd_col, ns_col, W, b_row)


def _out_body(p_ref, nd_ref, w_ref, b_ref, o_ref):
    agg = p_ref[0] + p_ref[1]
    x = agg * nd_ref[...]
    o_ref[...] = (
        jnp.dot(x, w_ref[...], preferred_element_type=jnp.float32) + b_ref[...]
    )


@jax.jit
def _layer_out(p, nd_col, W, b_row):
    blk = 1024
    return pl.pallas_call(
        _out_body,
        grid=(NPAD // blk,),
        in_specs=[
            pl.BlockSpec((NC, blk, D), lambda i: (0, i, 0)),
            pl.BlockSpec((blk, 1), lambda i: (i, 0)),
            pl.BlockSpec((D, D), lambda i: (0, 0)),
            pl.BlockSpec((1, D), lambda i: (0, 0)),
        ],
        out_specs=pl.BlockSpec((blk, D), lambda i: (i, 0)),
        out_shape=jax.ShapeDtypeStruct((NPAD, D), jnp.float32),
    )(p, nd_col, W, b_row)


@jax.jit
def kernel(feat, edge_index, labels, W1, b1, W2, b2):
    ei = edge_index.astype(jnp.int32)
    padidx = N + jnp.arange(EPAD - E, dtype=jnp.int32) % (NPAD - N)
    pad = jnp.stack([padidx, padidx])
    eip = jnp.concatenate([ei, pad], axis=1)
    src = eip[0]
    dst = eip[1]
    srcr = src.reshape(NT, 1, CHUNKS_PER_TILE, EDGE_CHUNK)
    dstr = dst.reshape(NT, 1, CHUNKS_PER_TILE, EDGE_CHUNK)
    sd = jnp.concatenate([srcr, dstr], axis=1)
    labr = labels.astype(jnp.int32).reshape(NT, LAB_CHUNKS_PER_TILE, LAB_CHUNK)
    featp = jnp.concatenate(
        [feat, jnp.zeros((NPAD - N, D), jnp.float32)], axis=0
    )
    zrows = jnp.zeros((ACC_ROWS_PER_TILE, D), jnp.float32)

    norm_s, norm_d = _degree_norms(src, dst)
    ns_col = norm_s.reshape(NPAD, 1)
    nd_col = norm_d.reshape(NPAD, 1)

    h1s = _scale(featp, ns_col)
    p1 = _aggregate(h1s, sd, zrows)
    h2s = _layer_mid(p1, nd_col, ns_col, W1, b1.reshape(1, D))
    p2 = _aggregate(h2s, sd, zrows)
    h2 = _layer_out(p2, nd_col, W2, b2.reshape(1, D))
    return _label_gather(h2, labr)
